# baseline (speedup 1.0000x reference)
import jax
import jax.numpy as jnp
from jax import lax
from jax.experimental import pallas as pl
from jax.experimental.pallas import tpu as pltpu, tpu_sc as plsc

_NC, _NS = 2, 16
_NW = _NC * _NS
_B = 16384
_DIM = 64
_OUTW = 3 * _DIM
_BPW = _B // _NW
_STREAM = 128
_NCHUNK = _BPW // _STREAM


def _gather_body(ent_hbm, rel_hbm, h_hbm, r_hbm, t_hbm, out_hbm,
                 hidx, ridx, tidx, hbuf, rbuf, tbuf,
                 gsem0, gsem1, gsem2, gsem3, isem, wsem):
    wid = lax.axis_index("s") * _NC + lax.axis_index("c")
    base = wid * _BPW
    i0 = pltpu.async_copy(h_hbm.at[pl.ds(base, _BPW)], hidx, isem)
    i1 = pltpu.async_copy(r_hbm.at[pl.ds(base, _BPW)], ridx, isem)
    i2 = pltpu.async_copy(t_hbm.at[pl.ds(base, _BPW)], tidx, isem)
    i0.wait()
    i1.wait()
    i2.wait()

    gsems = (gsem0, gsem1, gsem2, gsem3)
    gathers = []
    for c in range(_NCHUNK):
        s = pl.ds(c * _STREAM, _STREAM)
        gathers.append((
            pltpu.async_copy(ent_hbm.at[hidx.at[s]], hbuf.at[s], gsems[c]),
            pltpu.async_copy(rel_hbm.at[ridx.at[s]], rbuf.at[s], gsems[c]),
            pltpu.async_copy(ent_hbm.at[tidx.at[s]], tbuf.at[s], gsems[c]),
        ))

    writes = []
    for c in range(_NCHUNK):
        s = pl.ds(c * _STREAM, _STREAM)
        rows = pl.ds(base + c * _STREAM, _STREAM)
        for cp in gathers[c]:
            cp.wait()
        writes.append(pltpu.async_copy(
            hbuf.at[s], out_hbm.at[rows, pl.ds(0, _DIM)], wsem))
        writes.append(pltpu.async_copy(
            rbuf.at[s], out_hbm.at[rows, pl.ds(_DIM, _DIM)], wsem))
        writes.append(pltpu.async_copy(
            tbuf.at[s], out_hbm.at[rows, pl.ds(2 * _DIM, _DIM)], wsem))
    for cp in writes:
        cp.wait()


def kernel(entity_table, relation_table, h_idx, r_idx, t_idx, y):
    mesh = plsc.VectorSubcoreMesh(core_axis_name="c", subcore_axis_name="s")
    out = pl.kernel(
        _gather_body,
        out_type=jax.ShapeDtypeStruct((_B, _OUTW), jnp.float32),
        mesh=mesh,
        compiler_params=pltpu.CompilerParams(
            use_tc_tiling_on_sc=False,
            skip_device_barrier=True,
            disable_bounds_checks=True,
        ),
        scratch_types=[
            pltpu.VMEM((_BPW,), jnp.int32),
            pltpu.VMEM((_BPW,), jnp.int32),
            pltpu.VMEM((_BPW,), jnp.int32),
            pltpu.VMEM((_BPW, _DIM), jnp.float32),
            pltpu.VMEM((_BPW, _DIM), jnp.float32),
            pltpu.VMEM((_BPW, _DIM), jnp.float32),
            pltpu.SemaphoreType.DMA,
            pltpu.SemaphoreType.DMA,
            pltpu.SemaphoreType.DMA,
            pltpu.SemaphoreType.DMA,
            pltpu.SemaphoreType.DMA,
            pltpu.SemaphoreType.DMA,
        ],
    )(entity_table, relation_table,
      h_idx.astype(jnp.int32), r_idx.astype(jnp.int32), t_idx.astype(jnp.int32))
    return out.reshape(-1)


# speedup vs baseline: 1.0027x; 1.0027x over previous
import jax
import jax.numpy as jnp
from jax import lax
from jax.experimental import pallas as pl
from jax.experimental.pallas import tpu as pltpu, tpu_sc as plsc

_NC, _NS = 2, 16
_NW = _NC * _NS
_B = 16384
_DIM = 64
_OUTW = 3 * _DIM
_BPW = _B // _NW
_STREAM = 128
_NCHUNK = _BPW // _STREAM


def _gather_body(ent_hbm, rel_hbm, h_hbm, r_hbm, t_hbm, out_hbm,
                 hidx, ridx, tidx, hbuf, rbuf, tbuf,
                 gsem0, gsem1, gsem2, gsem3, isem, wsem):
    wid = lax.axis_index("s") * _NC + lax.axis_index("c")
    base = wid * _BPW
    i0 = pltpu.async_copy(h_hbm.at[pl.ds(base, _BPW)], hidx, isem)
    i1 = pltpu.async_copy(r_hbm.at[pl.ds(base, _BPW)], ridx, isem)
    i2 = pltpu.async_copy(t_hbm.at[pl.ds(base, _BPW)], tidx, isem)
    i0.wait()
    i1.wait()
    i2.wait()

    gsems = (gsem0, gsem1, gsem2, gsem3)
    gathers = []
    for c in range(_NCHUNK):
        s = pl.ds(c * _STREAM, _STREAM)
        gathers.append((
            pltpu.async_copy(ent_hbm.at[hidx.at[s]], hbuf.at[s], gsems[c]),
            pltpu.async_copy(rel_hbm.at[ridx.at[s]], rbuf.at[s], gsems[c]),
            pltpu.async_copy(ent_hbm.at[tidx.at[s]], tbuf.at[s], gsems[c]),
        ))

    writes = []
    for c in range(_NCHUNK):
        s = pl.ds(c * _STREAM, _STREAM)
        rows = pl.ds(base + c * _STREAM, _STREAM)
        for cp in gathers[c]:
            cp.wait()
        writes.append(pltpu.async_copy(
            hbuf.at[s], out_hbm.at[rows, pl.ds(0, _DIM)], wsem))
        writes.append(pltpu.async_copy(
            rbuf.at[s], out_hbm.at[rows, pl.ds(_DIM, _DIM)], wsem))
        writes.append(pltpu.async_copy(
            tbuf.at[s], out_hbm.at[rows, pl.ds(2 * _DIM, _DIM)], wsem))
    for cp in writes:
        cp.wait()


def kernel(entity_table, relation_table, h_idx, r_idx, t_idx, y):
    mesh = plsc.VectorSubcoreMesh(core_axis_name="c", subcore_axis_name="s")
    out = pl.kernel(
        _gather_body,
        out_type=jax.ShapeDtypeStruct((_B, _OUTW), jnp.float32),
        mesh=mesh,
        compiler_params=pltpu.CompilerParams(use_tc_tiling_on_sc=False),
        scratch_types=[
            pltpu.VMEM((_BPW,), jnp.int32),
            pltpu.VMEM((_BPW,), jnp.int32),
            pltpu.VMEM((_BPW,), jnp.int32),
            pltpu.VMEM((_BPW, _DIM), jnp.float32),
            pltpu.VMEM((_BPW, _DIM), jnp.float32),
            pltpu.VMEM((_BPW, _DIM), jnp.float32),
            pltpu.SemaphoreType.DMA,
            pltpu.SemaphoreType.DMA,
            pltpu.SemaphoreType.DMA,
            pltpu.SemaphoreType.DMA,
            pltpu.SemaphoreType.DMA,
            pltpu.SemaphoreType.DMA,
        ],
    )(entity_table, relation_table,
      h_idx.astype(jnp.int32), r_idx.astype(jnp.int32), t_idx.astype(jnp.int32))
    return out.reshape(-1)
